# Initial kernel scaffold; baseline (speedup 1.0000x reference)
import jax
import jax.numpy as jnp
from jax.experimental import pallas as pl


def kernel(user_id, item_id, category_id, context_id, table_0, table_1, table_2, table_3, W, b):
    raise NotImplementedError("write your pallas kernel here")


# speedup vs baseline: 1.6669x; 1.6669x over previous
import functools

import jax
import jax.numpy as jnp
from jax import lax
from jax.experimental import pallas as pl
from jax.experimental.pallas import tpu as pltpu
from jax.experimental.pallas import tpu_sc as plsc

B = 16384
V = 100000
D = 32
NF = 4
ROUTING_DIM = 128

_info = plsc.get_sparse_core_info()
_NC, _NS = _info.num_cores, _info.num_subcores
_NW = _NC * _NS
_BPW = B // _NW


def _sc_gather_body(i0, i1, i2, i3, t0, t1, t2, t3, out_hbm,
                    x0, x1, x2, x3, rows_v, sem):
    wid = lax.axis_index("s") * _NC + lax.axis_index("c")
    base = wid * _BPW
    idx_refs = (i0, i1, i2, i3)
    tab_refs = (t0, t1, t2, t3)
    idx_vs = (x0, x1, x2, x3)
    for f in range(NF):
        pltpu.sync_copy(idx_refs[f].at[pl.ds(base, _BPW)], idx_vs[f])
    copies = [
        pltpu.async_copy(tab_refs[f].at[idx_vs[f]], rows_v.at[f], sem)
        for f in range(NF)
    ]
    for f in range(NF):
        copies[f].wait()
        pltpu.sync_copy(rows_v.at[f], out_hbm.at[f, pl.ds(base, _BPW)])


@jax.jit
def _sc_gather(i0, i1, i2, i3, t0, t1, t2, t3):
    mesh = plsc.VectorSubcoreMesh(core_axis_name="c", subcore_axis_name="s")
    return pl.kernel(
        _sc_gather_body,
        mesh=mesh,
        out_type=jax.ShapeDtypeStruct((NF, B, D), jnp.float32),
        scratch_types=[
            pltpu.VMEM((_BPW,), jnp.int32),
            pltpu.VMEM((_BPW,), jnp.int32),
            pltpu.VMEM((_BPW,), jnp.int32),
            pltpu.VMEM((_BPW,), jnp.int32),
            pltpu.VMEM((NF, _BPW, D), jnp.float32),
            pltpu.SemaphoreType.DMA,
        ],
        compiler_params=pltpu.CompilerParams(use_tc_tiling_on_sc=False),
    )(i0, i1, i2, i3, t0, t1, t2, t3)


_BB = 2048


def _proj_body(e_ref, w_ref, b_ref, o_ref):
    x = jnp.concatenate([e_ref[f] for f in range(NF)], axis=1)
    o_ref[...] = jnp.dot(x, w_ref[...],
                         preferred_element_type=jnp.float32) + b_ref[...]


@jax.jit
def _tc_project(embs, W, b2d):
    return pl.pallas_call(
        _proj_body,
        grid=(B // _BB,),
        in_specs=[
            pl.BlockSpec((NF, _BB, D), lambda i: (0, i, 0)),
            pl.BlockSpec((NF * D, ROUTING_DIM), lambda i: (0, 0)),
            pl.BlockSpec((1, ROUTING_DIM), lambda i: (0, 0)),
        ],
        out_specs=pl.BlockSpec((_BB, ROUTING_DIM), lambda i: (i, 0)),
        out_shape=jax.ShapeDtypeStruct((B, ROUTING_DIM), jnp.float32),
    )(embs, W, b2d)


def kernel(user_id, item_id, category_id, context_id,
           table_0, table_1, table_2, table_3, W, b):
    embs = _sc_gather(user_id, item_id, category_id, context_id,
                      table_0, table_1, table_2, table_3)
    return _tc_project(embs, W, b.reshape(1, ROUTING_DIM))


# speedup vs baseline: 2.1631x; 1.2977x over previous
import functools

import jax
import jax.numpy as jnp
from jax import lax
from jax.experimental import pallas as pl
from jax.experimental.pallas import tpu as pltpu
from jax.experimental.pallas import tpu_sc as plsc

B = 16384
V = 100000
D = 32
NF = 4
ROUTING_DIM = 128

_info = plsc.get_sparse_core_info()
_NC, _NS = _info.num_cores, _info.num_subcores
_NW = _NC * _NS
_BPW = B // _NW
_CH = 128


def _sc_gather_body(i0, i1, i2, i3, t0, t1, t2, t3, out_hbm,
                    x0, x1, x2, x3, r0, r1, r2, r3, s0, s1, s2, s3):
    sems = (s0, s1, s2, s3)
    wid = lax.axis_index("s") * _NC + lax.axis_index("c")
    base = wid * _BPW
    idx_refs = (i0, i1, i2, i3)
    tab_refs = (t0, t1, t2, t3)
    idx_vs = (x0, x1, x2, x3)
    row_bufs = (r0, r1, r2, r3)
    for f in range(NF):
        pltpu.sync_copy(idx_refs[f].at[pl.ds(base, _BPW)], idx_vs[f])

    def chunk_step(c, _):
        cbase = c * _CH

        def issue16(cc, _):
            for f in range(NF):
                v = idx_vs[f][pl.ds(cbase + cc * 16, 16)]
                for j in range(16):
                    pltpu.async_copy(
                        tab_refs[f].at[pl.ds(v[j], 1), :],
                        row_bufs[f].at[pl.ds(cc * 16 + j, 1), :],
                        sems[f],
                    )
            return ()

        lax.fori_loop(0, _CH // 16, issue16, ())
        for f in range(NF):
            pltpu.make_async_copy(
                out_hbm.at[f, pl.ds(base + cbase, _CH)], row_bufs[f], sems[f]
            ).wait()
            pltpu.sync_copy(
                row_bufs[f], out_hbm.at[f, pl.ds(base + cbase, _CH)]
            )
        return ()

    lax.fori_loop(0, _BPW // _CH, chunk_step, ())


@jax.jit
def _sc_gather(i0, i1, i2, i3, t0, t1, t2, t3):
    mesh = plsc.VectorSubcoreMesh(core_axis_name="c", subcore_axis_name="s")
    return pl.kernel(
        _sc_gather_body,
        mesh=mesh,
        out_type=jax.ShapeDtypeStruct((NF, B, D), jnp.float32),
        scratch_types=[
            pltpu.VMEM((_BPW,), jnp.int32),
            pltpu.VMEM((_BPW,), jnp.int32),
            pltpu.VMEM((_BPW,), jnp.int32),
            pltpu.VMEM((_BPW,), jnp.int32),
            pltpu.VMEM((_CH, D), jnp.float32),
            pltpu.VMEM((_CH, D), jnp.float32),
            pltpu.VMEM((_CH, D), jnp.float32),
            pltpu.VMEM((_CH, D), jnp.float32),
            pltpu.SemaphoreType.DMA,
            pltpu.SemaphoreType.DMA,
            pltpu.SemaphoreType.DMA,
            pltpu.SemaphoreType.DMA,
        ],
    )(i0, i1, i2, i3, t0, t1, t2, t3)


_BB = 2048


def _proj_body(e_ref, w_ref, b_ref, o_ref):
    x = jnp.concatenate([e_ref[f] for f in range(NF)], axis=1)
    o_ref[...] = jnp.dot(x, w_ref[...],
                         preferred_element_type=jnp.float32) + b_ref[...]


@jax.jit
def _tc_project(embs, W, b2d):
    return pl.pallas_call(
        _proj_body,
        grid=(B // _BB,),
        in_specs=[
            pl.BlockSpec((NF, _BB, D), lambda i: (0, i, 0)),
            pl.BlockSpec((NF * D, ROUTING_DIM), lambda i: (0, 0)),
            pl.BlockSpec((1, ROUTING_DIM), lambda i: (0, 0)),
        ],
        out_specs=pl.BlockSpec((_BB, ROUTING_DIM), lambda i: (i, 0)),
        out_shape=jax.ShapeDtypeStruct((B, ROUTING_DIM), jnp.float32),
    )(embs, W, b2d)


def kernel(user_id, item_id, category_id, context_id,
           table_0, table_1, table_2, table_3, W, b):
    embs = _sc_gather(user_id, item_id, category_id, context_id,
                      table_0, table_1, table_2, table_3)
    return _tc_project(embs, W, b.reshape(1, ROUTING_DIM))


# speedup vs baseline: 3.7931x; 1.7535x over previous
import functools

import jax
import jax.numpy as jnp
from jax import lax
from jax.experimental import pallas as pl
from jax.experimental.pallas import tpu as pltpu
from jax.experimental.pallas import tpu_sc as plsc

B = 16384
V = 100000
D = 32
NF = 4
ROUTING_DIM = 128

_info = plsc.get_sparse_core_info()
_NC, _NS = _info.num_cores, _info.num_subcores
_NW = _NC * _NS
_IC = 4096


def _sc_gather_body(i0, i1, i2, i3, t0, t1, t2, t3, out_hbm,
                    row_v, idx_v, out_v):
    wid = lax.axis_index("s") * _NC + lax.axis_index("c")
    idx_refs = (i0, i1, i2, i3)
    tab_refs = (t0, t1, t2, t3)
    for f in range(NF):
        pltpu.sync_copy(tab_refs[f].at[wid], row_v)

        def idx_chunk(c, _):
            pltpu.sync_copy(idx_refs[f].at[pl.ds(c * _IC, _IC)], idx_v)

            def gather16(j, _):
                iv = idx_v[pl.ds(j * 16, 16)]
                vals = plsc.load_gather(row_v, [iv])
                out_v[pl.ds(c * _IC + j * 16, 16)] = vals
                return ()

            lax.fori_loop(0, _IC // 16, gather16, (), unroll=8)
            return ()

        lax.fori_loop(0, B // _IC, idx_chunk, ())
        pltpu.sync_copy(out_v, out_hbm.at[f * _NW + wid])


@jax.jit
def _sc_gather(i0, i1, i2, i3, t0, t1, t2, t3):
    mesh = plsc.VectorSubcoreMesh(core_axis_name="c", subcore_axis_name="s")
    return pl.kernel(
        _sc_gather_body,
        mesh=mesh,
        compiler_params=pltpu.CompilerParams(needs_layout_passes=False),
        out_type=jax.ShapeDtypeStruct((NF * D, B), jnp.float32),
        scratch_types=[
            pltpu.VMEM((V,), jnp.float32),
            pltpu.VMEM((_IC,), jnp.int32),
            pltpu.VMEM((B,), jnp.float32),
        ],
    )(i0, i1, i2, i3, t0, t1, t2, t3)


_BB = 2048


def _proj_body(e_ref, w_ref, b_ref, o_ref):
    o_ref[...] = jax.lax.dot_general(
        e_ref[...], w_ref[...], (((0,), (0,)), ((), ())),
        preferred_element_type=jnp.float32) + b_ref[...]


@jax.jit
def _tc_project(embsT, W, b2d):
    return pl.pallas_call(
        _proj_body,
        grid=(B // _BB,),
        in_specs=[
            pl.BlockSpec((NF * D, _BB), lambda i: (0, i)),
            pl.BlockSpec((NF * D, ROUTING_DIM), lambda i: (0, 0)),
            pl.BlockSpec((1, ROUTING_DIM), lambda i: (0, 0)),
        ],
        out_specs=pl.BlockSpec((_BB, ROUTING_DIM), lambda i: (i, 0)),
        out_shape=jax.ShapeDtypeStruct((B, ROUTING_DIM), jnp.float32),
    )(embsT, W, b2d)


def kernel(user_id, item_id, category_id, context_id,
           table_0, table_1, table_2, table_3, W, b):
    embsT = _sc_gather(user_id, item_id, category_id, context_id,
                       table_0.T, table_1.T, table_2.T, table_3.T)
    return _tc_project(embsT, W, b.reshape(1, ROUTING_DIM))
